# baseline (speedup 1.0000x reference)
import functools

import jax
import jax.numpy as jnp
from jax import lax
from jax.experimental import pallas as pl
from jax.experimental.pallas import tpu as pltpu
from jax.experimental.pallas import tpu_sc as plsc

_VOCAB = 1000000
_DIM = 64
_BATCH = 16384

_NC = 2
_NS = 16
_NW = _NC * _NS
_BPW = _BATCH // _NW


def _sc_gather_body(q_hbm, d_hbm, table_hbm, qout_hbm, dout_hbm,
                    qi_v, qr_v, di_v, dr_v, sq, sd):
    wid = lax.axis_index("s") * _NC + lax.axis_index("c")
    base = wid * _BPW
    pltpu.sync_copy(q_hbm.at[pl.ds(base, _BPW)], qi_v)
    pltpu.sync_copy(d_hbm.at[pl.ds(base, _BPW)], di_v)
    cq = pltpu.async_copy(table_hbm.at[qi_v], qr_v, sq)
    cd = pltpu.async_copy(table_hbm.at[di_v], dr_v, sd)
    cq.wait()
    cd.wait()
    pltpu.sync_copy(qr_v, qout_hbm.at[pl.ds(base, _BPW)])
    pltpu.sync_copy(dr_v, dout_hbm.at[pl.ds(base, _BPW)])


_sc_gather = pl.kernel(
    _sc_gather_body,
    out_type=(
        jax.ShapeDtypeStruct((_BATCH, _DIM), jnp.float32),
        jax.ShapeDtypeStruct((_BATCH, _DIM), jnp.float32),
    ),
    mesh=plsc.VectorSubcoreMesh(core_axis_name="c", subcore_axis_name="s"),
    scratch_types=[
        pltpu.VMEM((_BPW,), jnp.int32),
        pltpu.VMEM((_BPW, _DIM), jnp.float32),
        pltpu.VMEM((_BPW,), jnp.int32),
        pltpu.VMEM((_BPW, _DIM), jnp.float32),
        pltpu.SemaphoreType.DMA,
        pltpu.SemaphoreType.DMA,
    ],
    compiler_params=pltpu.CompilerParams(use_tc_tiling_on_sc=False),
)


def _proj_body(q_ref, d_ref, wq_ref, bq_ref, wd_ref, bd_ref, qo_ref, do_ref):
    q = q_ref[...]
    d = d_ref[...]
    yq = jnp.dot(q, wq_ref[...], preferred_element_type=jnp.float32) + bq_ref[...]
    yd = jnp.dot(d, wd_ref[...], preferred_element_type=jnp.float32) + bd_ref[...]
    sq = jnp.sum(yq * yq, axis=1, keepdims=True)
    sd = jnp.sum(yd * yd, axis=1, keepdims=True)
    qo_ref[...] = yq * lax.rsqrt(jnp.maximum(sq, 1e-24))
    do_ref[...] = yd * lax.rsqrt(jnp.maximum(sd, 1e-24))


def _tc_project(q_rows, d_rows, WqT, bq, WdT, bd):
    blk = 2048
    grid = (_BATCH // blk,)
    row_spec = pl.BlockSpec((blk, _DIM), lambda i: (i, 0))
    w_spec = pl.BlockSpec((_DIM, _DIM), lambda i: (0, 0))
    b_spec = pl.BlockSpec((1, _DIM), lambda i: (0, 0))
    return pl.pallas_call(
        _proj_body,
        grid=grid,
        in_specs=[row_spec, row_spec, w_spec, b_spec, w_spec, b_spec],
        out_specs=[row_spec, row_spec],
        out_shape=(
            jax.ShapeDtypeStruct((_BATCH, _DIM), jnp.float32),
            jax.ShapeDtypeStruct((_BATCH, _DIM), jnp.float32),
        ),
    )(q_rows, d_rows, WqT, bq, WdT, bd)


def kernel(queries, documents, table, Wq, bq, Wd, bd):
    q_rows, d_rows = _sc_gather(queries, documents, table)
    return _tc_project(q_rows, d_rows, Wq.T, bq.reshape(1, _DIM),
                       Wd.T, bd.reshape(1, _DIM))


# speedup vs baseline: 1.6652x; 1.6652x over previous
import jax
import jax.numpy as jnp
from jax import lax
from jax.experimental import pallas as pl
from jax.experimental.pallas import tpu as pltpu
from jax.experimental.pallas import tpu_sc as plsc

_VOCAB = 1000000
_DIM = 64
_BATCH = 16384

_NC = 2
_NS = 16
_NW = _NC * _NS
_BPW = _BATCH // _NW


def _sc_gather_body(q_hbm, d_hbm, table_hbm, qout_hbm, dout_hbm,
                    idx_v, rows_v, sem):
    wid = lax.axis_index("s") * _NC + lax.axis_index("c")
    base = wid * _BPW

    for idx_hbm, out_hbm in ((q_hbm, qout_hbm), (d_hbm, dout_hbm)):
        pltpu.sync_copy(idx_hbm.at[pl.ds(base, _BPW)], idx_v)

        def issue_body(b, _):
            iv = idx_v[pl.ds(b * 16, 16)]
            for j in range(16):
                r = iv[j]
                pltpu.async_copy(
                    table_hbm.at[pl.ds(r, 1)],
                    rows_v.at[pl.ds(b * 16 + j, 1)],
                    sem,
                )
            return 0
        lax.fori_loop(0, _BPW // 16, issue_body, 0)

        def drain_body(b, _):
            iv = idx_v[pl.ds(b * 16, 16)]
            for j in range(16):
                r = iv[j]
                pltpu.make_async_copy(
                    table_hbm.at[pl.ds(r, 1)],
                    rows_v.at[pl.ds(b * 16 + j, 1)],
                    sem,
                ).wait()
            return 0
        lax.fori_loop(0, _BPW // 16, drain_body, 0)

        pltpu.sync_copy(rows_v, out_hbm.at[pl.ds(base, _BPW)])


_sc_gather = pl.kernel(
    _sc_gather_body,
    out_type=(
        jax.ShapeDtypeStruct((_BATCH, _DIM), jnp.float32),
        jax.ShapeDtypeStruct((_BATCH, _DIM), jnp.float32),
    ),
    mesh=plsc.VectorSubcoreMesh(core_axis_name="c", subcore_axis_name="s"),
    scratch_types=[
        pltpu.VMEM((_BPW,), jnp.int32),
        pltpu.VMEM((_BPW, _DIM), jnp.float32),
        pltpu.SemaphoreType.DMA,
    ],
)


def _proj_body(q_ref, d_ref, wq_ref, bq_ref, wd_ref, bd_ref, qo_ref, do_ref):
    q = q_ref[...]
    d = d_ref[...]
    yq = jnp.dot(q, wq_ref[...], preferred_element_type=jnp.float32) + bq_ref[...]
    yd = jnp.dot(d, wd_ref[...], preferred_element_type=jnp.float32) + bd_ref[...]
    sq = jnp.sum(yq * yq, axis=1, keepdims=True)
    sd = jnp.sum(yd * yd, axis=1, keepdims=True)
    qo_ref[...] = yq * lax.rsqrt(jnp.maximum(sq, 1e-24))
    do_ref[...] = yd * lax.rsqrt(jnp.maximum(sd, 1e-24))


def _tc_project(q_rows, d_rows, WqT, bq, WdT, bd):
    blk = 2048
    grid = (_BATCH // blk,)
    row_spec = pl.BlockSpec((blk, _DIM), lambda i: (i, 0))
    w_spec = pl.BlockSpec((_DIM, _DIM), lambda i: (0, 0))
    b_spec = pl.BlockSpec((1, _DIM), lambda i: (0, 0))
    return pl.pallas_call(
        _proj_body,
        grid=grid,
        in_specs=[row_spec, row_spec, w_spec, b_spec, w_spec, b_spec],
        out_specs=[row_spec, row_spec],
        out_shape=(
            jax.ShapeDtypeStruct((_BATCH, _DIM), jnp.float32),
            jax.ShapeDtypeStruct((_BATCH, _DIM), jnp.float32),
        ),
    )(q_rows, d_rows, WqT, bq, WdT, bd)


def kernel(queries, documents, table, Wq, bq, Wd, bd):
    q_rows, d_rows = _sc_gather(queries, documents, table)
    return _tc_project(q_rows, d_rows, Wq.T, bq.reshape(1, _DIM),
                       Wd.T, bd.reshape(1, _DIM))
